# baseline (speedup 1.0000x reference)
import functools

import jax
import jax.numpy as jnp
from jax import lax
from jax.experimental import pallas as pl
from jax.experimental.pallas import tpu as pltpu
from jax.experimental.pallas import tpu_sc as plsc

_E = 64
_K = 8
_BLOCK_T = 2048
_NC = 2
_NS = 16
_NW = _NC * _NS


def _logits_body(x_ref, w_ref, e_ref):
    logits = lax.dot_general(
        w_ref[...],
        x_ref[...],
        dimension_numbers=(((0,), (1,)), ((), ())),
        preferred_element_type=jnp.float32,
    )
    m = jnp.max(logits, axis=0, keepdims=True)
    e_ref[...] = jnp.exp(logits - m)


def _make_topk(tokens):
    per = tokens // _NW
    ngroups = per // 16
    mesh = plsc.VectorSubcoreMesh(core_axis_name="c", subcore_axis_name="s")

    @functools.partial(
        pl.kernel,
        out_type=[
            jax.ShapeDtypeStruct((_K, tokens), jnp.float32),
            jax.ShapeDtypeStruct((_K, tokens), jnp.int32),
        ],
        mesh=mesh,
        scratch_types=[
            pltpu.VMEM((_E, per), jnp.float32),
            pltpu.VMEM((_K, per), jnp.float32),
            pltpu.VMEM((_K, per), jnp.int32),
        ],
    )
    def topk(e_hbm, w_hbm, i_hbm, tile, wv, iv):
        wid = lax.axis_index("s") * _NC + lax.axis_index("c")
        base = wid * per
        pltpu.sync_copy(e_hbm.at[:, pl.ds(base, per)], tile)
        lane = lax.broadcasted_iota(jnp.int32, (16,), 0)

        def group(g, carry):
            work = [tile[e, pl.ds(g * 16, 16)] for e in range(_E)]
            ssum = jnp.zeros((16,), jnp.float32)
            vals = []
            ids = []
            for _ in range(_K):
                vs = list(work)
                is_ = [jnp.full((16,), e, jnp.int32) for e in range(_E)]
                while len(vs) > 1:
                    nv, ni = [], []
                    for j in range(len(vs) // 2):
                        a, b = vs[2 * j], vs[2 * j + 1]
                        ai, bi = is_[2 * j], is_[2 * j + 1]
                        c = a >= b
                        nv.append(jnp.where(c, a, b))
                        ni.append(jnp.where(c, ai, bi))
                    vs, is_ = nv, ni
                m, mi = vs[0], is_[0]
                vals.append(m)
                ids.append(mi)
                ssum = ssum + m
                for e2 in range(_E):
                    work[e2] = jnp.where(mi == e2, -1.0, work[e2])
            for k in range(_K):
                wv[k, pl.ds(g * 16, 16)] = vals[k] / ssum
                iv[k, pl.ds(g * 16, 16)] = ids[k]
            return carry

        lax.fori_loop(0, ngroups, group, 0)
        pltpu.sync_copy(wv, w_hbm.at[:, pl.ds(base, per)])
        pltpu.sync_copy(iv, i_hbm.at[:, pl.ds(base, per)])

    return topk


@jax.jit
def kernel(x, W_router):
    tokens = x.shape[0]
    grid = (tokens // _BLOCK_T,)
    e_t = pl.pallas_call(
        _logits_body,
        grid=grid,
        in_specs=[
            pl.BlockSpec((_BLOCK_T, x.shape[1]), lambda t: (t, 0)),
            pl.BlockSpec((x.shape[1], _E), lambda t: (0, 0)),
        ],
        out_specs=pl.BlockSpec((_E, _BLOCK_T), lambda t: (0, t)),
        out_shape=jax.ShapeDtypeStruct((_E, tokens), jnp.float32),
    )(x, W_router)
    weights_t, ids_t = _make_topk(tokens)(e_t)
    return weights_t.T, ids_t.T


# speedup vs baseline: 2.2498x; 2.2498x over previous
import jax
import jax.numpy as jnp
from jax.experimental import pallas as pl

_NUM_EXPERTS = 64
_TOP_K = 8
_BLOCK_T = 2048


def _router_body(x_ref, w_ref, wout_ref, iout_ref):
    logits = jax.lax.dot_general(
        w_ref[...],
        x_ref[...],
        dimension_numbers=(((0,), (1,)), ((), ())),
        preferred_element_type=jnp.float32,
    )
    bt = logits.shape[1]
    m = jnp.max(logits, axis=0, keepdims=True)
    e = jnp.exp(logits - m)
    eidx = jax.lax.broadcasted_iota(jnp.int32, (_NUM_EXPERTS, bt), 0)

    work = e
    vals = []
    ids = []
    for _ in range(_TOP_K):
        cur = jnp.max(work, axis=0, keepdims=True)
        idx = jnp.min(
            jnp.where(work == cur, eidx, _NUM_EXPERTS), axis=0, keepdims=True
        )
        vals.append(cur)
        ids.append(idx)
        work = jnp.where(eidx == idx, -1.0, work)

    v = jnp.concatenate(vals, axis=0)
    i = jnp.concatenate(ids, axis=0)
    wout_ref[...] = v / jnp.sum(v, axis=0, keepdims=True)
    iout_ref[...] = i


@jax.jit
def kernel(x, W_router):
    tokens = x.shape[0]
    grid = (tokens // _BLOCK_T,)
    weights_t, ids_t = pl.pallas_call(
        _router_body,
        grid=grid,
        in_specs=[
            pl.BlockSpec((_BLOCK_T, x.shape[1]), lambda t: (t, 0)),
            pl.BlockSpec((x.shape[1], _NUM_EXPERTS), lambda t: (0, 0)),
        ],
        out_specs=[
            pl.BlockSpec((_TOP_K, _BLOCK_T), lambda t: (0, t)),
            pl.BlockSpec((_TOP_K, _BLOCK_T), lambda t: (0, t)),
        ],
        out_shape=[
            jax.ShapeDtypeStruct((_TOP_K, tokens), jnp.float32),
            jax.ShapeDtypeStruct((_TOP_K, tokens), jnp.int32),
        ],
    )(x, W_router)
    return weights_t.T, ids_t.T
